# Initial kernel scaffold; baseline (speedup 1.0000x reference)
import jax
import jax.numpy as jnp
from jax.experimental import pallas as pl


def kernel(x, edge_index, edge_weight, W_gat, att_src, att_dst, att_edge, W_edge, b_gat, W_sage_l, b_sage, W_sage_r, W_lin1, b_lin1, W_lin2, b_lin2):
    raise NotImplementedError("write your pallas kernel here")


# speedup vs baseline: 1.0059x; 1.0059x over previous
import functools

import jax
import jax.numpy as jnp
from jax.experimental import pallas as pl
from jax.experimental.pallas import tpu as pltpu

N = 10000
E = 160000
C_IN = 256
C_H = 256
HEADS = 4
HC = HEADS * C_H
BN = 1000


def _p1_body(x_ref, wg_ref, asrc_ref, adst_ref, h_ref, asd_ref):
    h = jnp.dot(x_ref[...], wg_ref[...], preferred_element_type=jnp.float32)
    h_ref[...] = h
    a_s = jnp.dot(h, asrc_ref[...], preferred_element_type=jnp.float32)
    a_d = jnp.dot(h, adst_ref[...], preferred_element_type=jnp.float32)
    asd_ref[...] = jnp.concatenate([a_s, a_d], axis=1)


def _phase1(x, W_gat, A_src, A_dst):
    grid = (N // BN,)
    return pl.pallas_call(
        _p1_body,
        grid=grid,
        in_specs=[
            pl.BlockSpec((BN, C_IN), lambda i: (i, 0)),
            pl.BlockSpec((C_IN, HC), lambda i: (0, 0)),
            pl.BlockSpec((HC, HEADS), lambda i: (0, 0)),
            pl.BlockSpec((HC, HEADS), lambda i: (0, 0)),
        ],
        out_specs=[
            pl.BlockSpec((BN, HC), lambda i: (i, 0)),
            pl.BlockSpec((BN, 2 * HEADS), lambda i: (i, 0)),
        ],
        out_shape=[
            jax.ShapeDtypeStruct((N, HC), jnp.float32),
            jax.ShapeDtypeStruct((N, 2 * HEADS), jnp.float32),
        ],
    )(x, W_gat, A_src, A_dst)


def _p4_body(pre_ref, b_ref, wl_ref, wr_ref, y_ref, z_ref):
    o1 = jnp.maximum(pre_ref[...] + b_ref[...], 0.0)
    y_ref[...] = jnp.dot(o1, wl_ref[...], preferred_element_type=jnp.float32)
    z_ref[...] = jnp.dot(o1, wr_ref[...], preferred_element_type=jnp.float32)


def _phase4(out1pre, b_gat, W_sage_l, W_sage_r):
    grid = (N // BN,)
    return pl.pallas_call(
        _p4_body,
        grid=grid,
        in_specs=[
            pl.BlockSpec((BN, HC), lambda i: (i, 0)),
            pl.BlockSpec((1, HC), lambda i: (0, 0)),
            pl.BlockSpec((HC, C_H), lambda i: (0, 0)),
            pl.BlockSpec((HC, C_H), lambda i: (0, 0)),
        ],
        out_specs=[
            pl.BlockSpec((BN, C_H), lambda i: (i, 0)),
            pl.BlockSpec((BN, C_H), lambda i: (i, 0)),
        ],
        out_shape=[
            jax.ShapeDtypeStruct((N, C_H), jnp.float32),
            jax.ShapeDtypeStruct((N, C_H), jnp.float32),
        ],
    )(out1pre, b_gat.reshape(1, HC), W_sage_l, W_sage_r)


def _p6_body(agg_ref, s_ref, z_ref, bs_ref, w1_ref, b1_ref, w2_ref, b2_ref,
             out_ref):
    deg = jnp.clip(s_ref[:, 4:5], 1.0, None)
    o2 = jnp.maximum(agg_ref[...] / deg + bs_ref[...] + z_ref[...], 0.0)
    o3 = jnp.maximum(
        jnp.dot(o2, w1_ref[...], preferred_element_type=jnp.float32)
        + b1_ref[...], 0.0)
    out_ref[...] = (
        jnp.dot(o3, w2_ref[...], preferred_element_type=jnp.float32)
        + b2_ref[...])


def _phase6(aggsum, s_tab, z, b_sage, W_lin1, b_lin1, W_lin2, b_lin2):
    grid = (N // BN,)
    return pl.pallas_call(
        _p6_body,
        grid=grid,
        in_specs=[
            pl.BlockSpec((BN, C_H), lambda i: (i, 0)),
            pl.BlockSpec((BN, 8), lambda i: (i, 0)),
            pl.BlockSpec((BN, C_H), lambda i: (i, 0)),
            pl.BlockSpec((1, C_H), lambda i: (0, 0)),
            pl.BlockSpec((C_H, C_H // 2), lambda i: (0, 0)),
            pl.BlockSpec((1, C_H // 2), lambda i: (0, 0)),
            pl.BlockSpec((C_H // 2, 1), lambda i: (0, 0)),
            pl.BlockSpec((1, 1), lambda i: (0, 0)),
        ],
        out_specs=pl.BlockSpec((BN, 1), lambda i: (i, 0)),
        out_shape=jax.ShapeDtypeStruct((N, 1), jnp.float32),
    )(aggsum, s_tab, z, b_sage.reshape(1, C_H), W_lin1,
      b_lin1.reshape(1, C_H // 2), W_lin2, b_lin2.reshape(1, 1))


def _edge_stats_jnp(src, dst, ew, asd, c4):
    a = asd[src, :4] + asd[dst, 4:] + ew[:, None] * c4[None, :]
    a = jnp.maximum(a, 0.2 * a)
    g = jnp.exp(a)
    ones = jnp.ones((E, 1), jnp.float32)
    upd = jnp.concatenate([g, ones, jnp.zeros((E, 3), jnp.float32)], axis=1)
    s_tab = jax.ops.segment_sum(upd, dst, num_segments=N)
    return g, s_tab


def _gat_agg_jnp(src, dst, g, s_tab, h):
    att = g / (s_tab[dst, :4] + 1e-16)
    w = jnp.repeat(att, C_H, axis=1)
    return jax.ops.segment_sum(h[src] * w, dst, num_segments=N)


def _sage_agg_jnp(src, dst, y):
    return jax.ops.segment_sum(y[src], dst, num_segments=N)


def kernel(x, edge_index, edge_weight, W_gat, att_src, att_dst, att_edge,
           W_edge, b_gat, W_sage_l, b_sage, W_sage_r, W_lin1, b_lin1,
           W_lin2, b_lin2):
    src = edge_index[0]
    dst = edge_index[1]
    eye = jnp.eye(HEADS, dtype=jnp.float32)
    A_src = (att_src[:, :, None] * eye[:, None, :]).reshape(HC, HEADS)
    A_dst = (att_dst[:, :, None] * eye[:, None, :]).reshape(HC, HEADS)
    c4 = jnp.sum(W_edge.reshape(HEADS, C_H) * att_edge, axis=1)

    h, asd = _phase1(x, W_gat, A_src, A_dst)
    g, s_tab = _edge_stats_jnp(src, dst, edge_weight, asd, c4)
    out1pre = _gat_agg_jnp(src, dst, g, s_tab, h)
    y, z = _phase4(out1pre, b_gat, W_sage_l, W_sage_r)
    aggsum = _sage_agg_jnp(src, dst, y)
    return _phase6(aggsum, s_tab, z, b_sage, W_lin1, b_lin1, W_lin2, b_lin2)
